# Initial kernel scaffold; baseline (speedup 1.0000x reference)
import jax
import jax.numpy as jnp
from jax.experimental import pallas as pl


def kernel(X, ppr_scores, ppr_idx, W1, W2, W3, W4):
    raise NotImplementedError("write your pallas kernel here")


# speedup vs baseline: 2.9722x; 2.9722x over previous
import jax
import jax.numpy as jnp
from jax import lax
from jax.experimental import pallas as pl
from jax.experimental.pallas import tpu as pltpu

N = 320000
F_IN = 128
H = 128
C = 64
B = 10000

R = 1024
NBLK = -(-N // R)
NP = NBLK * R
W = 128
ACC_ROWS = B + 2 * W


def _body(s0_ref, idx_ref, sc_ref, x_ref, w1_ref, w2_ref, w3_ref, w4_ref,
          out_ref, acc_ref):
    pid = pl.program_id(0)

    @pl.when(pid == 0)
    def _init():
        acc_ref[...] = jnp.zeros((ACC_ROWS, H), jnp.float32)

    x = x_ref[...]
    h = jnp.maximum(jnp.dot(x, w1_ref[...], preferred_element_type=jnp.float32), 0.0)
    e = jnp.dot(h, w2_ref[...], preferred_element_type=jnp.float32)

    seg = idx_ref[0]
    sc = sc_ref[0]
    s0 = s0_ref[pid]
    local = seg - s0
    nwin = jnp.max(local) // W + 1
    iota = lax.broadcasted_iota(jnp.int32, (W, R), 0)

    def win(k, carry):
        base = k * W
        oh = jnp.where(local == base + iota, sc, 0.0)
        contrib = lax.dot_general(oh, e, (((1,), (0,)), ((), ())),
                                  preferred_element_type=jnp.float32)
        start = s0 + base
        acc_ref[pl.ds(start, W), :] += contrib
        return carry

    lax.fori_loop(0, nwin, win, 0)

    @pl.when(pid == NBLK - 1)
    def _final():
        p = acc_ref[0:B, :]
        h2 = jnp.maximum(jnp.dot(p, w3_ref[...], preferred_element_type=jnp.float32), 0.0)
        out_ref[...] = jnp.dot(h2, w4_ref[...], preferred_element_type=jnp.float32)


def kernel(X, ppr_scores, ppr_idx, W1, W2, W3, W4):
    pad = NP - N
    Xp = jnp.pad(X, ((0, pad), (0, 0)))
    scp = jnp.pad(ppr_scores, (0, pad))
    idxp = jnp.pad(ppr_idx, (0, pad), mode="edge")
    s0s = idxp[::R]
    idx3 = idxp.reshape(NBLK, 1, R)
    sc3 = scp.reshape(NBLK, 1, R)

    grid_spec = pltpu.PrefetchScalarGridSpec(
        num_scalar_prefetch=1,
        grid=(NBLK,),
        in_specs=[
            pl.BlockSpec((1, 1, R), lambda i, s0s: (i, 0, 0)),
            pl.BlockSpec((1, 1, R), lambda i, s0s: (i, 0, 0)),
            pl.BlockSpec((R, F_IN), lambda i, s0s: (i, 0)),
            pl.BlockSpec((F_IN, H), lambda i, s0s: (0, 0)),
            pl.BlockSpec((H, H), lambda i, s0s: (0, 0)),
            pl.BlockSpec((H, H), lambda i, s0s: (0, 0)),
            pl.BlockSpec((H, C), lambda i, s0s: (0, 0)),
        ],
        out_specs=pl.BlockSpec((B, C), lambda i, s0s: (0, 0)),
        scratch_shapes=[pltpu.VMEM((ACC_ROWS, H), jnp.float32)],
    )

    return pl.pallas_call(
        _body,
        grid_spec=grid_spec,
        out_shape=jax.ShapeDtypeStruct((B, C), jnp.float32),
        compiler_params=pltpu.CompilerParams(
            dimension_semantics=("arbitrary",),
        ),
    )(s0s, idx3, sc3, Xp, W1, W2, W3, W4)


# speedup vs baseline: 7.5714x; 2.5474x over previous
import jax
import jax.numpy as jnp
from jax import lax
from jax.experimental import pallas as pl
from jax.experimental.pallas import tpu as pltpu

N = 320000
F_IN = 128
H = 128
C = 64
B = 10000

R = 2560
NBLK = N // R
W = 128
ACC_ROWS = B + 2 * W


def _body(s0_ref, idx_ref, sc_ref, x_ref, w1_ref, w2_ref, w3_ref, w4_ref,
          out_ref, acc_ref):
    pid = pl.program_id(0)

    @pl.when(pid == 0)
    def _init():
        acc_ref[...] = jnp.zeros((ACC_ROWS, H), jnp.float32)

    x = x_ref[...].astype(jnp.bfloat16)
    h = jnp.maximum(
        jnp.dot(x, w1_ref[...].astype(jnp.bfloat16),
                preferred_element_type=jnp.float32), 0.0
    ).astype(jnp.bfloat16)

    seg = idx_ref[0]
    sc = sc_ref[0]
    s0 = s0_ref[pid]
    local = seg - s0
    nwin = jnp.max(local) // W + 1
    iota = lax.broadcasted_iota(jnp.int32, (W, R), 0)

    def win(k, carry):
        base = k * W
        oh = jnp.where(local == base + iota, sc, 0.0).astype(jnp.bfloat16)
        contrib = lax.dot_general(oh, h, (((1,), (0,)), ((), ())),
                                  preferred_element_type=jnp.float32)
        start = s0 + base
        acc_ref[pl.ds(start, W), :] += contrib
        return carry

    lax.fori_loop(0, nwin, win, 0)

    @pl.when(pid == NBLK - 1)
    def _final():
        w23 = jnp.dot(w2_ref[...].astype(jnp.bfloat16),
                      w3_ref[...].astype(jnp.bfloat16),
                      preferred_element_type=jnp.float32).astype(jnp.bfloat16)
        p = acc_ref[0:B, :].astype(jnp.bfloat16)
        h2 = jnp.maximum(
            jnp.dot(p, w23, preferred_element_type=jnp.float32), 0.0
        ).astype(jnp.bfloat16)
        out_ref[...] = jnp.dot(h2, w4_ref[...].astype(jnp.bfloat16),
                               preferred_element_type=jnp.float32)


def kernel(X, ppr_scores, ppr_idx, W1, W2, W3, W4):
    s0s = ppr_idx[::R]
    idx3 = ppr_idx.reshape(NBLK, 1, R)
    sc3 = ppr_scores.reshape(NBLK, 1, R)

    grid_spec = pltpu.PrefetchScalarGridSpec(
        num_scalar_prefetch=1,
        grid=(NBLK,),
        in_specs=[
            pl.BlockSpec((1, 1, R), lambda i, s0s: (i, 0, 0)),
            pl.BlockSpec((1, 1, R), lambda i, s0s: (i, 0, 0)),
            pl.BlockSpec((R, F_IN), lambda i, s0s: (i, 0)),
            pl.BlockSpec((F_IN, H), lambda i, s0s: (0, 0)),
            pl.BlockSpec((H, H), lambda i, s0s: (0, 0)),
            pl.BlockSpec((H, H), lambda i, s0s: (0, 0)),
            pl.BlockSpec((H, C), lambda i, s0s: (0, 0)),
        ],
        out_specs=pl.BlockSpec((B, C), lambda i, s0s: (0, 0)),
        scratch_shapes=[pltpu.VMEM((ACC_ROWS, H), jnp.float32)],
    )

    return pl.pallas_call(
        _body,
        grid_spec=grid_spec,
        out_shape=jax.ShapeDtypeStruct((B, C), jnp.float32),
        compiler_params=pltpu.CompilerParams(
            dimension_semantics=("arbitrary",),
        ),
    )(s0s, idx3, sc3, X, W1, W2, W3, W4)


# speedup vs baseline: 9.6365x; 1.2728x over previous
import jax
import jax.numpy as jnp
from jax import lax
from jax.experimental import pallas as pl
from jax.experimental.pallas import tpu as pltpu

N = 320000
F_IN = 128
H = 128
C = 64
B = 10000

R = 6400
NBLK = N // R
W = 128
ACC_ROWS = B + 2 * W


def _body(s0_ref, idx_ref, sc_ref, x_ref, w1_ref, w2_ref, w3_ref, w4_ref,
          out_ref, acc_ref):
    pid = pl.program_id(0)

    @pl.when(pid == 0)
    def _init():
        acc_ref[...] = jnp.zeros((ACC_ROWS, H), jnp.float32)

    x = x_ref[...].astype(jnp.bfloat16)
    h = jnp.maximum(
        jnp.dot(x, w1_ref[...].astype(jnp.bfloat16),
                preferred_element_type=jnp.float32), 0.0
    ).astype(jnp.bfloat16)

    seg = idx_ref[0]
    sc = sc_ref[0]
    s0 = s0_ref[pid]
    local = seg - s0
    nwin = jnp.max(local) // W + 1
    iota = lax.broadcasted_iota(jnp.int32, (W, R), 0)

    def win(k, carry):
        base = k * W
        oh = jnp.where(local == base + iota, sc, 0.0).astype(jnp.bfloat16)
        contrib = lax.dot_general(oh, h, (((1,), (0,)), ((), ())),
                                  preferred_element_type=jnp.float32)
        start = s0 + base
        acc_ref[pl.ds(start, W), :] += contrib
        return carry

    lax.fori_loop(0, nwin, win, 0)

    @pl.when(pid == NBLK - 1)
    def _final():
        w23 = jnp.dot(w2_ref[...].astype(jnp.bfloat16),
                      w3_ref[...].astype(jnp.bfloat16),
                      preferred_element_type=jnp.float32).astype(jnp.bfloat16)
        p = acc_ref[0:B, :].astype(jnp.bfloat16)
        h2 = jnp.maximum(
            jnp.dot(p, w23, preferred_element_type=jnp.float32), 0.0
        ).astype(jnp.bfloat16)
        out_ref[...] = jnp.dot(h2, w4_ref[...].astype(jnp.bfloat16),
                               preferred_element_type=jnp.float32)


def kernel(X, ppr_scores, ppr_idx, W1, W2, W3, W4):
    s0s = ppr_idx[::R]
    idx3 = ppr_idx.reshape(NBLK, 1, R)
    sc3 = ppr_scores.reshape(NBLK, 1, R)

    grid_spec = pltpu.PrefetchScalarGridSpec(
        num_scalar_prefetch=1,
        grid=(NBLK,),
        in_specs=[
            pl.BlockSpec((1, 1, R), lambda i, s0s: (i, 0, 0)),
            pl.BlockSpec((1, 1, R), lambda i, s0s: (i, 0, 0)),
            pl.BlockSpec((R, F_IN), lambda i, s0s: (i, 0)),
            pl.BlockSpec((F_IN, H), lambda i, s0s: (0, 0)),
            pl.BlockSpec((H, H), lambda i, s0s: (0, 0)),
            pl.BlockSpec((H, H), lambda i, s0s: (0, 0)),
            pl.BlockSpec((H, C), lambda i, s0s: (0, 0)),
        ],
        out_specs=pl.BlockSpec((B, C), lambda i, s0s: (0, 0)),
        scratch_shapes=[pltpu.VMEM((ACC_ROWS, H), jnp.float32)],
    )

    return pl.pallas_call(
        _body,
        grid_spec=grid_spec,
        out_shape=jax.ShapeDtypeStruct((B, C), jnp.float32),
        compiler_params=pltpu.CompilerParams(
            dimension_semantics=("arbitrary",),
        ),
    )(s0s, idx3, sc3, X, W1, W2, W3, W4)
